# baseline (speedup 1.0000x reference)
import functools

import jax
import jax.numpy as jnp
from jax import lax
from jax.experimental import pallas as pl
from jax.experimental.pallas import tpu as pltpu
from jax.experimental.pallas import tpu_sc as plsc

N = 10000
E = 320000
D = 128
NC = 2
NS = 16
NW = NC * NS
EPW = E // NW
CH = 80
NCHUNK = EPW // CH
RPT_A = 632
RPT_LAST = N - 15 * RPT_A
RPT = N // NS
ZR = 125
LG = D // 16


def _spmm_body(h_hbm, ed_hbm, out_hbm,
               eb_v, rows_v, zbuf_v, acc_sh, sem_g, sem_i, sem_s):
    c = lax.axis_index("c")
    s = lax.axis_index("s")
    wid = c * NS + s

    zero16 = jnp.zeros((16,), jnp.float32)

    def zrow(i, carry):
        for j in range(LG):
            zbuf_v[i, pl.ds(j * 16, 16)] = zero16
        return carry

    lax.fori_loop(0, ZR, zrow, 0)

    def zcopy(k, carry):
        pltpu.sync_copy(zbuf_v, acc_sh.at[pl.ds(s * RPT + k * ZR, ZR), :])
        return carry

    lax.fori_loop(0, RPT // ZR, zcopy, 0)
    plsc.subcore_barrier()

    cbase = wid * NCHUNK

    def fire_idx(t, q):
        pltpu.async_copy(ed_hbm.at[cbase + t], eb_v.at[q], sem_i)

    def wait_idx(q):
        pltpu.make_async_copy(ed_hbm.at[0], eb_v.at[q], sem_i).wait()

    def fire_gather(t, p):
        pltpu.async_copy(h_hbm.at[eb_v.at[(t & 3), 0]], rows_v.at[p],
                         sem_g.at[p])

    def wait_gather(p):
        pltpu.make_async_copy(h_hbm.at[pl.ds(0, CH), :], rows_v.at[p],
                              sem_g.at[p]).wait()

    def fire_scatter(t, p):
        pltpu.async_copy(rows_v.at[p], acc_sh.at[eb_v.at[(t & 3), 1]],
                         sem_s, add=True)

    def wait_scatter():
        pltpu.make_async_copy(rows_v.at[0], acc_sh.at[pl.ds(0, CH), :],
                              sem_s).wait()

    pltpu.sync_copy(ed_hbm.at[cbase], eb_v.at[0])
    fire_gather(0, 0)
    fire_idx(1, 1)
    wait_idx(1)
    fire_gather(1, 1)
    fire_idx(2, 2)

    def chunk_body(t, carry):
        p = lax.rem(t, 3)
        q = t & 3

        @pl.when(t + 2 < NCHUNK)
        def _prefetch():
            wait_idx((t + 2) & 3)

            @pl.when(t >= 1)
            def _():
                wait_scatter()

            fire_gather(t + 2, lax.rem(t + 2, 3))

        @pl.when(t + 3 < NCHUNK)
        def _():
            fire_idx(t + 3, (t + 3) & 3)

        wait_gather(p)

        def scale_row(g, cc):
            for i in range(8):
                r = g * 8 + i
                wb = plsc.bitcast(
                    plsc.load_gather(eb_v.at[q, 2],
                                     [jnp.full((16,), r, jnp.int32)]),
                    jnp.float32)
                for j in range(LG):
                    rows_v[p, r, pl.ds(j * 16, 16)] = (
                        rows_v[p, r, pl.ds(j * 16, 16)] * wb)
            return cc

        lax.fori_loop(0, CH // 8, scale_row, 0)
        fire_scatter(t, p)
        return carry

    lax.fori_loop(0, NCHUNK, chunk_body, 0)
    wait_scatter()
    wait_scatter()
    wait_scatter()
    plsc.subcore_barrier()

    @pl.when(s < NS - 1)
    def _wr_main():
        off = pl.multiple_of(s * RPT_A, 8)
        pltpu.sync_copy(acc_sh.at[pl.ds(off, RPT_A), :],
                        out_hbm.at[c, pl.ds(off, RPT_A), :])

    @pl.when(s == NS - 1)
    def _wr_last():
        pltpu.sync_copy(acc_sh.at[pl.ds((NS - 1) * RPT_A, RPT_LAST), :],
                        out_hbm.at[c, pl.ds((NS - 1) * RPT_A, RPT_LAST), :])


def _spmm_partials(h, ed):
    mesh = plsc.VectorSubcoreMesh(core_axis_name="c", subcore_axis_name="s",
                                  num_cores=NC, num_subcores=NS)
    kfn = pl.kernel(
        _spmm_body,
        out_type=jax.ShapeDtypeStruct((NC, N, D), jnp.float32),
        mesh=mesh,
        scratch_types=[
            pltpu.VMEM((4, 3, CH), jnp.int32),
            pltpu.VMEM((3, CH, D), jnp.float32),
            pltpu.VMEM((ZR, D), jnp.float32),
            pltpu.VMEM_SHARED((N, D), jnp.float32),
            pltpu.SemaphoreType.DMA((3,)),
            pltpu.SemaphoreType.DMA,
            pltpu.SemaphoreType.DMA,
        ],
        compiler_params=pltpu.CompilerParams(needs_layout_passes=False),
    )
    return kfn(h, ed)


_BLK = 1000


def _mm1_body(x_ref, w_ref, o_ref):
    o_ref[...] = jnp.dot(x_ref[...], w_ref[...],
                         preferred_element_type=jnp.float32)


def _mm1(x, W1):
    return pl.pallas_call(
        _mm1_body,
        out_shape=jax.ShapeDtypeStruct((N, D), jnp.float32),
        grid=(N // _BLK,),
        in_specs=[pl.BlockSpec((_BLK, D), lambda i: (i, 0)),
                  pl.BlockSpec((D, D), lambda i: (0, 0))],
        out_specs=pl.BlockSpec((_BLK, D), lambda i: (i, 0)),
    )(x, W1)


def _mm2_body(p_ref, b_ref, w_ref, o_ref):
    h = p_ref[0] + p_ref[1] + b_ref[...]
    h = jnp.maximum(h, 0.0)
    o_ref[...] = jnp.dot(h, w_ref[...], preferred_element_type=jnp.float32)


def _relu_mm2(p, b1, W2):
    return pl.pallas_call(
        _mm2_body,
        out_shape=jax.ShapeDtypeStruct((N, D), jnp.float32),
        grid=(N // _BLK,),
        in_specs=[pl.BlockSpec((NC, _BLK, D), lambda i: (0, i, 0)),
                  pl.BlockSpec((1, D), lambda i: (0, 0)),
                  pl.BlockSpec((D, D), lambda i: (0, 0))],
        out_specs=pl.BlockSpec((_BLK, D), lambda i: (i, 0)),
    )(p, b1.reshape(1, D), W2)


def _lsm_body(p_ref, b_ref, o_ref):
    z = p_ref[0] + p_ref[1] + b_ref[...]
    m = jnp.max(z, axis=1, keepdims=True)
    e = jnp.exp(z - m)
    lse = jnp.log(jnp.sum(e, axis=1, keepdims=True)) + m
    o_ref[...] = z - lse


def _log_softmax(p, b2):
    return pl.pallas_call(
        _lsm_body,
        out_shape=jax.ShapeDtypeStruct((N, D), jnp.float32),
        grid=(N // _BLK,),
        in_specs=[pl.BlockSpec((NC, _BLK, D), lambda i: (0, i, 0)),
                  pl.BlockSpec((1, D), lambda i: (0, 0))],
        out_specs=pl.BlockSpec((_BLK, D), lambda i: (i, 0)),
    )(p, b2.reshape(1, D))


def kernel(x, edge_index, edge_weight, W1, b1, W2, b2):
    wbits = lax.bitcast_convert_type(edge_weight, jnp.int32)
    ed = jnp.stack([edge_index[0], edge_index[1], wbits], axis=0)
    ed = ed.reshape(3, E // CH, CH).transpose(1, 0, 2)
    h1 = _mm1(x, W1)
    p1 = _spmm_partials(h1, ed)
    h2 = _relu_mm2(p1, b1, W2)
    p2 = _spmm_partials(h2, ed)
    return _log_softmax(p2, b2)


# speedup vs baseline: 2.4179x; 2.4179x over previous
import functools

import jax
import jax.numpy as jnp
from jax import lax
from jax.experimental import pallas as pl
from jax.experimental.pallas import tpu as pltpu
from jax.experimental.pallas import tpu_sc as plsc

N = 10000
E = 320000
D = 128
NC = 2
NS = 16
NW = NC * NS
EPW = E // NW
CH = 80
NCHUNK = EPW // CH
RPT_A = 632
RPT_LAST = N - 15 * RPT_A
RPT = N // NS
ZR = 125
LG = D // 16


def _spmm_body(h_hbm, ed_hbm, out_hbm,
               eb_v, rows_v, zbuf_v, acc_sh, sem_g, sem_i, sem_s):
    c = lax.axis_index("c")
    s = lax.axis_index("s")
    wid = c * NS + s

    zero16 = jnp.zeros((16,), jnp.float32)

    def zrow(i, carry):
        for j in range(LG):
            zbuf_v[i, pl.ds(j * 16, 16)] = zero16
        return carry

    lax.fori_loop(0, ZR, zrow, 0)

    def zcopy(k, carry):
        pltpu.sync_copy(zbuf_v, acc_sh.at[pl.ds(s * RPT + k * ZR, ZR), :])
        return carry

    lax.fori_loop(0, RPT // ZR, zcopy, 0)
    plsc.subcore_barrier()

    cbase = wid * NCHUNK

    def fire_idx(t, q):
        pltpu.async_copy(ed_hbm.at[cbase + t], eb_v.at[q], sem_i)

    def wait_idx(q):
        pltpu.make_async_copy(ed_hbm.at[0], eb_v.at[q], sem_i).wait()

    def fire_gather(t, p):
        pltpu.async_copy(h_hbm.at[eb_v.at[(t & 3), 0]], rows_v.at[p],
                         sem_g.at[p])

    def wait_gather(p):
        pltpu.make_async_copy(h_hbm.at[pl.ds(0, CH), :], rows_v.at[p],
                              sem_g.at[p]).wait()

    def fire_scatter(t, p):
        pltpu.async_copy(rows_v.at[p], acc_sh.at[eb_v.at[(t & 3), 1]],
                         sem_s, add=True)

    def wait_scatter():
        pltpu.make_async_copy(rows_v.at[0], acc_sh.at[pl.ds(0, CH), :],
                              sem_s).wait()

    pltpu.sync_copy(ed_hbm.at[cbase], eb_v.at[0])
    fire_gather(0, 0)
    fire_idx(1, 1)

    def chunk_body(t, carry):
        p = t & 1
        q = t & 3

        @pl.when(t + 1 < NCHUNK)
        def _prefetch():
            wait_idx((t + 1) & 3)

            @pl.when(t >= 1)
            def _():
                wait_scatter()

            fire_gather(t + 1, 1 - p)

        @pl.when(t + 2 < NCHUNK)
        def _():
            fire_idx(t + 2, (t + 2) & 3)

        wait_gather(p)

        def scale_row(g, cc):
            for i in range(8):
                r = g * 8 + i
                wb = plsc.bitcast(
                    plsc.load_gather(eb_v.at[q, 2],
                                     [jnp.full((16,), r, jnp.int32)]),
                    jnp.float32)
                for j in range(LG):
                    rows_v[p, r, pl.ds(j * 16, 16)] = (
                        rows_v[p, r, pl.ds(j * 16, 16)] * wb)
            return cc

        lax.fori_loop(0, CH // 8, scale_row, 0)
        fire_scatter(t, p)
        return carry

    lax.fori_loop(0, NCHUNK, chunk_body, 0)
    wait_scatter()
    wait_scatter()
    plsc.subcore_barrier()

    @pl.when(s < NS - 1)
    def _wr_main():
        off = pl.multiple_of(s * RPT_A, 8)
        pltpu.sync_copy(acc_sh.at[pl.ds(off, RPT_A), :],
                        out_hbm.at[c, pl.ds(off, RPT_A), :])

    @pl.when(s == NS - 1)
    def _wr_last():
        pltpu.sync_copy(acc_sh.at[pl.ds((NS - 1) * RPT_A, RPT_LAST), :],
                        out_hbm.at[c, pl.ds((NS - 1) * RPT_A, RPT_LAST), :])


def _spmm_partials(h, ed):
    mesh = plsc.VectorSubcoreMesh(core_axis_name="c", subcore_axis_name="s",
                                  num_cores=NC, num_subcores=NS)
    kfn = pl.kernel(
        _spmm_body,
        out_type=jax.ShapeDtypeStruct((NC, N, D), jnp.float32),
        mesh=mesh,
        scratch_types=[
            pltpu.VMEM((4, 3, CH), jnp.int32),
            pltpu.VMEM((2, CH, D), jnp.float32),
            pltpu.VMEM((ZR, D), jnp.float32),
            pltpu.VMEM_SHARED((N, D), jnp.float32),
            pltpu.SemaphoreType.DMA((2,)),
            pltpu.SemaphoreType.DMA,
            pltpu.SemaphoreType.DMA,
        ],
        compiler_params=pltpu.CompilerParams(needs_layout_passes=False),
    )
    return kfn(h, ed)


_BLK = 1000


def _mm1_body(x_ref, w_ref, o_ref):
    o_ref[...] = jnp.dot(x_ref[...], w_ref[...],
                         preferred_element_type=jnp.float32)


def _mm1(x, W1):
    return pl.pallas_call(
        _mm1_body,
        out_shape=jax.ShapeDtypeStruct((N, D), jnp.float32),
        grid=(N // _BLK,),
        in_specs=[pl.BlockSpec((_BLK, D), lambda i: (i, 0)),
                  pl.BlockSpec((D, D), lambda i: (0, 0))],
        out_specs=pl.BlockSpec((_BLK, D), lambda i: (i, 0)),
    )(x, W1)


def _mm2_body(p_ref, b_ref, w_ref, o_ref):
    h = p_ref[0] + p_ref[1] + b_ref[...]
    h = jnp.maximum(h, 0.0)
    o_ref[...] = jnp.dot(h, w_ref[...], preferred_element_type=jnp.float32)


def _relu_mm2(p, b1, W2):
    return pl.pallas_call(
        _mm2_body,
        out_shape=jax.ShapeDtypeStruct((N, D), jnp.float32),
        grid=(N // _BLK,),
        in_specs=[pl.BlockSpec((NC, _BLK, D), lambda i: (0, i, 0)),
                  pl.BlockSpec((1, D), lambda i: (0, 0)),
                  pl.BlockSpec((D, D), lambda i: (0, 0))],
        out_specs=pl.BlockSpec((_BLK, D), lambda i: (i, 0)),
    )(p, b1.reshape(1, D), W2)


def _lsm_body(p_ref, b_ref, o_ref):
    z = p_ref[0] + p_ref[1] + b_ref[...]
    m = jnp.max(z, axis=1, keepdims=True)
    e = jnp.exp(z - m)
    lse = jnp.log(jnp.sum(e, axis=1, keepdims=True)) + m
    o_ref[...] = z - lse


def _log_softmax(p, b2):
    return pl.pallas_call(
        _lsm_body,
        out_shape=jax.ShapeDtypeStruct((N, D), jnp.float32),
        grid=(N // _BLK,),
        in_specs=[pl.BlockSpec((NC, _BLK, D), lambda i: (0, i, 0)),
                  pl.BlockSpec((1, D), lambda i: (0, 0))],
        out_specs=pl.BlockSpec((_BLK, D), lambda i: (i, 0)),
    )(p, b2.reshape(1, D))


def kernel(x, edge_index, edge_weight, W1, b1, W2, b2):
    wbits = lax.bitcast_convert_type(edge_weight, jnp.int32)
    ed = jnp.stack([edge_index[0], edge_index[1], wbits], axis=0)
    ed = ed.reshape(3, E // CH, CH).transpose(1, 0, 2)
    h1 = _mm1(x, W1)
    p1 = _spmm_partials(h1, ed)
    h2 = _relu_mm2(p1, b1, W2)
    p2 = _spmm_partials(h2, ed)
    return _log_softmax(p2, b2)
